# baseline (speedup 1.0000x reference)
import functools

import jax
import jax.numpy as jnp
from jax import lax
from jax.experimental import pallas as pl
from jax.experimental.pallas import tpu as pltpu
from jax.experimental.pallas import tpu_sc as plsc

_N = 10000
_E = 320000
_F_IN = 128
_DIM = 16
_OUT = 128

_NC = 2
_NS = 16
_NW = _NC * _NS
_CHUNK = 128
_NCHUNKS = _E // _CHUNK
_MAXC = _NCHUNKS // _NW + 1
_N_PAD = 10240
_ROWS_PER_SUB = _N_PAD // _NS


_NBUF = 8


def _segsum_body(y_hbm, ei_hbm, z_hbm, out_hbm, acc, ytab, srcv, dstv,
                 rows, gsems, ssems):
    c = lax.axis_index("c")
    s = lax.axis_index("s")
    wid = c * _NS + s

    cstart = (wid * _NCHUNKS) // _NW
    cnt = ((wid + 1) * _NCHUNKS) // _NW - cstart

    sl = pl.ds(s * _ROWS_PER_SUB, _ROWS_PER_SUB)
    d1 = pltpu.async_copy(y_hbm.at[sl], ytab.at[sl], gsems.at[0])
    d2 = pltpu.async_copy(z_hbm.at[sl], acc.at[sl], gsems.at[1])
    d3 = pltpu.async_copy(ei_hbm.at[0, pl.ds(cstart, _MAXC)], srcv,
                          gsems.at[2])
    d4 = pltpu.async_copy(ei_hbm.at[1, pl.ds(cstart, _MAXC)], dstv,
                          gsems.at[3])
    d1.wait()
    d2.wait()
    d3.wait()
    d4.wait()
    plsc.subcore_barrier()

    def _buf(b):
        return rows.at[b]

    def _gather(j, b):
        pltpu.async_copy(ytab.at[srcv.at[j]], _buf(b), gsems.at[b])

    def _wait_gather(b):
        pltpu.make_async_copy(ytab.at[srcv.at[0]], _buf(b),
                              gsems.at[b]).wait()

    def _scatter(j, b):
        pltpu.async_copy(_buf(b), acc.at[dstv.at[j]], ssems.at[b], add=True)

    def _wait_scatter(b):
        pltpu.make_async_copy(_buf(b), acc.at[dstv.at[0]], ssems.at[b]).wait()

    for j in range(_NBUF):
        _gather(j, j)

    def _visit(j, carry):
        b = lax.rem(j, _NBUF)
        bp = lax.rem(j + _NBUF - 1, _NBUF)

        @pl.when(j >= 1)
        def _():
            _wait_scatter(bp)

            @pl.when(j - 1 + _NBUF < cnt)
            def _():
                _gather(j - 1 + _NBUF, bp)

        _wait_gather(b)
        _scatter(j, b)
        return carry

    lax.fori_loop(0, cnt, _visit, 0)
    _wait_scatter(lax.rem(cnt - 1, _NBUF))
    plsc.subcore_barrier()

    pltpu.sync_copy(acc.at[pl.ds(s * _ROWS_PER_SUB, _ROWS_PER_SUB)],
                    out_hbm.at[c, pl.ds(s * _ROWS_PER_SUB, _ROWS_PER_SUB)])


_segsum = functools.partial(
    pl.kernel,
    out_type=jax.ShapeDtypeStruct((_NC, _N_PAD, _DIM), jnp.float32),
    mesh=plsc.VectorSubcoreMesh(core_axis_name="c", subcore_axis_name="s"),
    scratch_types=[
        pltpu.VMEM_SHARED((_N_PAD, _DIM), jnp.float32),
        pltpu.VMEM_SHARED((_N_PAD, _DIM), jnp.float32),
        pltpu.VMEM((_MAXC, _CHUNK), jnp.int32),
        pltpu.VMEM((_MAXC, _CHUNK), jnp.int32),
        pltpu.VMEM((_NBUF, _CHUNK, _DIM), jnp.float32),
        pltpu.SemaphoreType.DMA((_NBUF,)),
        pltpu.SemaphoreType.DMA((_NBUF,)),
    ],
    compiler_params=pltpu.CompilerParams(use_tc_tiling_on_sc=False),
)(_segsum_body)


_PROWS = _N_PAD // 8
_PROWS_REAL = _N // 8


def _proj0_body(h_ref, w_ref, y_ref):
    y_ref[0:_N, :] = jnp.dot(h_ref[...], w_ref[...],
                             preferred_element_type=jnp.float32)
    y_ref[_N:_N_PAD, :] = jnp.zeros((_N_PAD - _N, _DIM), jnp.float32)


_proj0 = pl.pallas_call(
    _proj0_body,
    out_shape=jax.ShapeDtypeStruct((_N_PAD, _DIM), jnp.float32),
)


def _mid_body(parts_ref, y_ref, b1_ref, w2_ref, b2_ref, w1n_ref,
              ynext_ref, csum_ref):
    s = parts_ref[0] + parts_ref[1]
    hdn = jnp.maximum(y_ref[...] + s + b1_ref[...], 0.0)
    x = jnp.maximum(
        jnp.dot(hdn, w2_ref[...], preferred_element_type=jnp.float32)
        + b2_ref[...], 0.0)
    ynext_ref[...] = jnp.dot(x, w1n_ref[...],
                             preferred_element_type=jnp.float32)
    csum_ref[...] = jnp.sum(x[0:_PROWS_REAL], axis=0, keepdims=True)


_mid = pl.pallas_call(
    _mid_body,
    out_shape=(
        jax.ShapeDtypeStruct((_PROWS, 128), jnp.float32),
        jax.ShapeDtypeStruct((1, 128), jnp.float32),
    ),
)


def _final_body(parts_ref, y_ref, b1_ref, w2_ref, b2_ref, csums_ref,
                heads_ref, out_ref):
    s = parts_ref[0] + parts_ref[1]
    hdn = jnp.maximum(y_ref[...] + s + b1_ref[...], 0.0)
    x = jnp.maximum(
        jnp.dot(hdn, w2_ref[...], preferred_element_type=jnp.float32)
        + b2_ref[...], 0.0)
    c5 = jnp.sum(x[0:_PROWS_REAL], axis=0, keepdims=True)
    allc = jnp.concatenate([csums_ref[...], c5], axis=0) * (1.0 / _N)
    fold = allc[:, 0:_DIM]
    for a in range(1, 8):
        fold = fold + allc[:, a * _DIM:(a + 1) * _DIM]
    total = jnp.zeros((1, _OUT), jnp.float32)
    for l in range(5):
        total = total + jnp.dot(fold[l:l + 1, :], heads_ref[l],
                                preferred_element_type=jnp.float32)
    out_ref[...] = jnp.tanh(total)


_final = pl.pallas_call(
    _final_body,
    out_shape=jax.ShapeDtypeStruct((1, _OUT), jnp.float32),
)


def kernel(h, edge_index, conv_params, head_params):
    ei = edge_index.reshape(2, _NCHUNKS, _CHUNK)
    zrows = jnp.zeros((_N_PAD, _DIM), jnp.float32)

    eye8 = jnp.eye(8, dtype=jnp.float32)
    b1t = [jnp.tile(p[1], 8) for p in conv_params]
    w2b = [jnp.kron(eye8, p[2]) for p in conv_params]
    b2t = [jnp.tile(p[3], 8) for p in conv_params]
    w1b = [jnp.kron(eye8, p[0]) for p in conv_params]
    heads = jnp.stack(head_params, axis=0)

    y_sc = _proj0(h, conv_params[0][0])
    y_p = jnp.reshape(y_sc, (_PROWS, 128))

    csums = []
    out = None
    for l in range(5):
        parts = _segsum(y_sc, ei, zrows)
        parts_p = jnp.reshape(parts, (_NC, _PROWS, 128))
        if l < 4:
            y_p, cs = _mid(parts_p, y_p, b1t[l], w2b[l], b2t[l], w1b[l + 1])
            y_sc = jnp.reshape(y_p, (_N_PAD, _DIM))
            csums.append(cs)
        else:
            csums4 = jnp.concatenate(csums, axis=0)
            out = _final(parts_p, y_p, b1t[l], w2b[l], b2t[l], csums4,
                         heads)
    return out


# speedup vs baseline: 1.0536x; 1.0536x over previous
import functools

import jax
import jax.numpy as jnp
from jax import lax
from jax.experimental import pallas as pl
from jax.experimental.pallas import tpu as pltpu
from jax.experimental.pallas import tpu_sc as plsc

_N = 10000
_E = 320000
_F_IN = 128
_DIM = 16
_OUT = 128

_NC = 2
_NS = 16
_NW = _NC * _NS
_CHUNK = 256
_NCHUNKS = _E // _CHUNK
_GRP = 1
_NGRP = _NCHUNKS // _GRP
_MAXG = _NGRP // _NW + 1
_MAXC = _MAXG * _GRP
_N_PAD = 10240
_ROWS_PER_SUB = _N_PAD // _NS


_NBUF = 4


def _segsum_body(y_hbm, ei_hbm, z_hbm, out_hbm, acc, ytab, srcv, dstv,
                 rows, gsems, ssems):
    c = lax.axis_index("c")
    s = lax.axis_index("s")
    wid = c * _NS + s

    gstart = (wid * _NGRP) // _NW
    cnt = ((wid + 1) * _NGRP) // _NW - gstart
    cstart = gstart * _GRP

    sl = pl.ds(s * _ROWS_PER_SUB, _ROWS_PER_SUB)
    d1 = pltpu.async_copy(y_hbm.at[sl], ytab.at[sl], gsems.at[0])
    d2 = pltpu.async_copy(z_hbm.at[sl], acc.at[sl], gsems.at[1])
    d3 = pltpu.async_copy(ei_hbm.at[0, pl.ds(cstart, _MAXC)], srcv,
                          gsems.at[2])
    d4 = pltpu.async_copy(ei_hbm.at[1, pl.ds(cstart, _MAXC)], dstv,
                          gsems.at[3])
    d1.wait()
    d2.wait()
    d3.wait()
    d4.wait()
    plsc.subcore_barrier()

    def _buf(b):
        return rows.at[b]

    def _idx(ref, j):
        return ref.at[j]

    def _gather(j, b):
        pltpu.async_copy(ytab.at[_idx(srcv, j)], _buf(b), gsems.at[b])

    def _wait_gather(b):
        pltpu.make_async_copy(ytab.at[_idx(srcv, 0)], _buf(b),
                              gsems.at[b]).wait()

    def _scatter(j, b):
        pltpu.async_copy(_buf(b), acc.at[_idx(dstv, j)], ssems.at[b],
                         add=True)

    def _wait_scatter(b):
        pltpu.make_async_copy(_buf(b), acc.at[_idx(dstv, 0)],
                              ssems.at[b]).wait()

    for j in range(_NBUF):
        _gather(j, j)

    def _visit(j, carry):
        b = lax.rem(j, _NBUF)
        bp = lax.rem(j + _NBUF - 1, _NBUF)

        @pl.when(j >= 1)
        def _():
            _wait_scatter(bp)

            @pl.when(j - 1 + _NBUF < cnt)
            def _():
                _gather(j - 1 + _NBUF, bp)

        _wait_gather(b)
        _scatter(j, b)
        return carry

    lax.fori_loop(0, cnt, _visit, 0)
    _wait_scatter(lax.rem(cnt - 1, _NBUF))
    plsc.subcore_barrier()

    pltpu.sync_copy(acc.at[pl.ds(s * _ROWS_PER_SUB, _ROWS_PER_SUB)],
                    out_hbm.at[c, pl.ds(s * _ROWS_PER_SUB, _ROWS_PER_SUB)])


_segsum = functools.partial(
    pl.kernel,
    out_type=jax.ShapeDtypeStruct((_NC, _N_PAD, _DIM), jnp.float32),
    mesh=plsc.VectorSubcoreMesh(core_axis_name="c", subcore_axis_name="s"),
    scratch_types=[
        pltpu.VMEM_SHARED((_N_PAD, _DIM), jnp.float32),
        pltpu.VMEM_SHARED((_N_PAD, _DIM), jnp.float32),
        pltpu.VMEM((_MAXC, _CHUNK), jnp.int32),
        pltpu.VMEM((_MAXC, _CHUNK), jnp.int32),
        pltpu.VMEM((_NBUF, _CHUNK, _DIM), jnp.float32),
        pltpu.SemaphoreType.DMA((_NBUF,)),
        pltpu.SemaphoreType.DMA((_NBUF,)),
    ],
    compiler_params=pltpu.CompilerParams(use_tc_tiling_on_sc=False),
)(_segsum_body)


_PROWS = _N_PAD // 8
_PROWS_REAL = _N // 8


def _proj0_body(h_ref, w_ref, y_ref):
    y_ref[0:_N, :] = jnp.dot(h_ref[...], w_ref[...],
                             preferred_element_type=jnp.float32)
    y_ref[_N:_N_PAD, :] = jnp.zeros((_N_PAD - _N, _DIM), jnp.float32)


_proj0 = pl.pallas_call(
    _proj0_body,
    out_shape=jax.ShapeDtypeStruct((_N_PAD, _DIM), jnp.float32),
)


def _mid_body(parts_ref, y_ref, b1_ref, w2_ref, b2_ref, w1n_ref,
              ynext_ref, csum_ref):
    s = parts_ref[0] + parts_ref[1]
    hdn = jnp.maximum(y_ref[...] + s + b1_ref[...], 0.0)
    x = jnp.maximum(
        jnp.dot(hdn, w2_ref[...], preferred_element_type=jnp.float32)
        + b2_ref[...], 0.0)
    ynext_ref[...] = jnp.dot(x, w1n_ref[...],
                             preferred_element_type=jnp.float32)
    csum_ref[...] = jnp.sum(x[0:_PROWS_REAL], axis=0, keepdims=True)


_mid = pl.pallas_call(
    _mid_body,
    out_shape=(
        jax.ShapeDtypeStruct((_PROWS, 128), jnp.float32),
        jax.ShapeDtypeStruct((1, 128), jnp.float32),
    ),
)


def _final_body(parts_ref, y_ref, b1_ref, w2_ref, b2_ref, csums_ref,
                heads_ref, out_ref):
    s = parts_ref[0] + parts_ref[1]
    hdn = jnp.maximum(y_ref[...] + s + b1_ref[...], 0.0)
    x = jnp.maximum(
        jnp.dot(hdn, w2_ref[...], preferred_element_type=jnp.float32)
        + b2_ref[...], 0.0)
    c5 = jnp.sum(x[0:_PROWS_REAL], axis=0, keepdims=True)
    allc = jnp.concatenate([csums_ref[...], c5], axis=0) * (1.0 / _N)
    fold = allc[:, 0:_DIM]
    for a in range(1, 8):
        fold = fold + allc[:, a * _DIM:(a + 1) * _DIM]
    total = jnp.zeros((1, _OUT), jnp.float32)
    for l in range(5):
        total = total + jnp.dot(fold[l:l + 1, :], heads_ref[l],
                                preferred_element_type=jnp.float32)
    out_ref[...] = jnp.tanh(total)


_final = pl.pallas_call(
    _final_body,
    out_shape=jax.ShapeDtypeStruct((1, _OUT), jnp.float32),
)


def kernel(h, edge_index, conv_params, head_params):
    ei = edge_index.reshape(2, _NCHUNKS, _CHUNK)
    zrows = jnp.zeros((_N_PAD, _DIM), jnp.float32)

    eye8 = jnp.eye(8, dtype=jnp.float32)
    b1t = [jnp.tile(p[1], 8) for p in conv_params]
    w2b = [jnp.kron(eye8, p[2]) for p in conv_params]
    b2t = [jnp.tile(p[3], 8) for p in conv_params]
    w1b = [jnp.kron(eye8, p[0]) for p in conv_params]
    heads = jnp.stack(head_params, axis=0)

    y_sc = _proj0(h, conv_params[0][0])
    y_p = jnp.reshape(y_sc, (_PROWS, 128))

    csums = []
    out = None
    for l in range(5):
        parts = _segsum(y_sc, ei, zrows)
        parts_p = jnp.reshape(parts, (_NC, _PROWS, 128))
        if l < 4:
            y_p, cs = _mid(parts_p, y_p, b1t[l], w2b[l], b2t[l], w1b[l + 1])
            y_sc = jnp.reshape(y_p, (_N_PAD, _DIM))
            csums.append(cs)
        else:
            csums4 = jnp.concatenate(csums, axis=0)
            out = _final(parts_p, y_p, b1t[l], w2b[l], b2t[l], csums4,
                         heads)
    return out
